# baseline (speedup 1.0000x reference)
import functools
import math

import jax
import jax.numpy as jnp
from jax import lax
from jax.experimental import pallas as pl
from jax.experimental.pallas import tpu as pltpu
from jax.experimental.pallas import tpu_sc as plsc

T = 16
N = 10000
E = 5000
D = 128

NC = 2
NS = 16
L = 16

EDGES_PER_TILE = 320
E_PAD = NS * EDGES_PER_TILE
CHUNK = 80
NCHUNK = EDGES_PER_TILE // CHUNK
ROWS_PER_TILE = 640
N_Y = NS * ROWS_PER_TILE
N_PAD = N_Y + L
SLICES_PER_SC = T // NC
RCHUNK = 80
NR = ROWS_PER_TILE // RCHUNK
NEG = -3.0e38


def _pe_table(t, d_model):
    position = jnp.arange(t, dtype=jnp.float32)[:, None]
    div_term = jnp.exp(
        jnp.arange(0, d_model, 2, dtype=jnp.float32) * (-math.log(10000.0) / d_model))
    pe = jnp.zeros((t, d_model), dtype=jnp.float32)
    pe = pe.at[:, 0::2].set(jnp.sin(position * div_term))
    pe = pe.at[:, 1::2].set(jnp.cos(position * div_term))
    return pe


def _matmul_body(q_ref, w_ref, y_ref):
    y_ref[...] = jnp.dot(q_ref[...], w_ref[...],
                         preferred_element_type=jnp.float32)


def _compute_y(q, w):
    blk = 1280
    return pl.pallas_call(
        _matmul_body,
        grid=(N_Y // blk,),
        in_specs=[pl.BlockSpec((blk, D), lambda i: (i, 0)),
                  pl.BlockSpec((D, D), lambda i: (0, 0))],
        out_specs=pl.BlockSpec((blk, D), lambda i: (i, 0)),
        out_shape=jax.ShapeDtypeStruct((N_Y, D), jnp.float32),
    )(q, w)


_mesh = plsc.VectorSubcoreMesh(core_axis_name="c", subcore_axis_name="s")


@functools.partial(
    pl.kernel,
    mesh=_mesh,
    out_type=jax.ShapeDtypeStruct((T * D,), jnp.float32),
    scratch_types=[
        pltpu.VMEM_SHARED((N_PAD, D), jnp.float32),
        pltpu.VMEM_SHARED((NS, D), jnp.float32),
        pltpu.VMEM((NCHUNK, CHUNK), jnp.int32),
        pltpu.VMEM((NCHUNK, CHUNK), jnp.int32),
        pltpu.VMEM((2, CHUNK, D), jnp.float32),
        pltpu.VMEM((2, RCHUNK, D), jnp.float32),
        pltpu.VMEM((D,), jnp.float32),
        pltpu.VMEM((NS, D), jnp.float32),
        pltpu.VMEM((D,), jnp.float32),
        pltpu.VMEM((D,), jnp.float32),
        pltpu.SemaphoreType.DMA,
        pltpu.SemaphoreType.DMA,
        pltpu.SemaphoreType.DMA,
    ],
)
def _encoder_sc(y_hbm, src_hbm, dst_hbm, pe_hbm, out_hbm,
                agg_sh, partials_sh, src_idx_v, dst_idx_v, rows_v,
                stripe_v, vec_v, partials_v, pe_v, out_v, gsem, msem, isem):
    c = lax.axis_index("c")
    s = lax.axis_index("s")
    rbase = s * ROWS_PER_TILE

    t0 = c * SLICES_PER_SC
    pltpu.sync_copy(src_hbm.at[t0, s], src_idx_v)
    pltpu.sync_copy(dst_hbm.at[t0, s], dst_idx_v)
    pltpu.async_copy(y_hbm.at[src_idx_v.at[0]], rows_v.at[0], gsem)
    pltpu.sync_copy(y_hbm.at[pl.ds(rbase, ROWS_PER_TILE)],
                    agg_sh.at[pl.ds(rbase, ROWS_PER_TILE)])
    plsc.subcore_barrier()

    def slice_body(k, carry):
        t = c * SLICES_PER_SC + k
        with jax.named_scope("scatter_phase"):
            for j in range(NCHUNK):
                pltpu.make_async_copy(y_hbm.at[src_idx_v.at[j]],
                                      rows_v.at[j % 2], gsem).wait()
                if j + 1 < NCHUNK:
                    pltpu.async_copy(y_hbm.at[src_idx_v.at[j + 1]],
                                     rows_v.at[(j + 1) % 2], gsem)
                pltpu.sync_copy(rows_v.at[j % 2],
                                agg_sh.at[dst_idx_v.at[j]], add=True)
        with jax.named_scope("barrier1"):
            plsc.subcore_barrier()
        pltpu.async_copy(agg_sh.at[pl.ds(rbase, RCHUNK)], stripe_v.at[0], msem)

        @pl.when(k + 1 < SLICES_PER_SC)
        def _():
            tn = t + 1
            pltpu.sync_copy(src_hbm.at[tn, s], src_idx_v)
            pltpu.sync_copy(dst_hbm.at[tn, s], dst_idx_v)
            pltpu.async_copy(y_hbm.at[src_idx_v.at[0]], rows_v.at[0], gsem)

        sc_max = jax.named_scope("max_phase")
        sc_max.__enter__()
        acc = tuple(jnp.full((L,), NEG, jnp.float32) for _ in range(D // L))
        for cb in range(NR):
            pltpu.make_async_copy(
                agg_sh.at[pl.ds(rbase + cb * RCHUNK, RCHUNK)],
                stripe_v.at[cb % 2], msem).wait()
            if cb + 1 < NR:
                pltpu.async_copy(
                    agg_sh.at[pl.ds(rbase + (cb + 1) * RCHUNK, RCHUNK)],
                    stripe_v.at[(cb + 1) % 2], msem)

            @pl.when(k + 1 < SLICES_PER_SC)
            def _():
                pltpu.async_copy(
                    y_hbm.at[pl.ds(rbase + cb * RCHUNK, RCHUNK)],
                    agg_sh.at[pl.ds(rbase + cb * RCHUNK, RCHUNK)], isem)

            buf = cb % 2

            def row_body(r, a):
                return tuple(
                    jnp.maximum(a[j], stripe_v[buf, r, pl.ds(j * L, L)])
                    for j in range(D // L))

            acc = lax.fori_loop(0, RCHUNK, row_body, acc)

        @pl.when(k + 1 < SLICES_PER_SC)
        def _():
            for cb in range(NR):
                pltpu.make_async_copy(
                    y_hbm.at[pl.ds(rbase + cb * RCHUNK, RCHUNK)],
                    agg_sh.at[pl.ds(rbase + cb * RCHUNK, RCHUNK)],
                    isem).wait()

        sc_max.__exit__(None, None, None)
        with jax.named_scope("epilogue"):
            for j in range(D // L):
                vec_v[pl.ds(j * L, L)] = jnp.maximum(acc[j], 0.0)
            pltpu.sync_copy(vec_v, partials_sh.at[s])
            plsc.subcore_barrier()

        @pl.when(s == 0)
        def _():
            pltpu.sync_copy(partials_sh, partials_v)
            pltpu.sync_copy(pe_hbm.at[pl.ds(t * D, D)], pe_v)
            for j in range(D // L):
                m = partials_v[0, pl.ds(j * L, L)]
                for i in range(1, NS):
                    m = jnp.maximum(m, partials_v[i, pl.ds(j * L, L)])
                out_v[pl.ds(j * L, L)] = m + pe_v[pl.ds(j * L, L)]
            pltpu.sync_copy(out_v, out_hbm.at[pl.ds(t * D, D)])

        plsc.subcore_barrier()
        return carry

    lax.fori_loop(0, SLICES_PER_SC, slice_body, 0)


def kernel(q_embeddings, W, edge_index):
    q_pad = jnp.concatenate(
        [q_embeddings, jnp.zeros((N_Y - N, D), jnp.float32)], axis=0)
    y = _compute_y(q_pad, W)
    ei = edge_index.astype(jnp.int32)
    src = ei[:, 0, :]
    dst = ei[:, 1, :]
    pad_src = jnp.zeros((T, E_PAD - E), jnp.int32)
    pad_dst = jnp.full((T, E_PAD - E), N_Y, jnp.int32)
    src_p = jnp.concatenate([src, pad_src], axis=1).reshape(T, NS, NCHUNK, CHUNK)
    dst_p = jnp.concatenate([dst, pad_dst], axis=1).reshape(T, NS, NCHUNK, CHUNK)
    pe = _pe_table(T, D).reshape(T * D)
    out = _encoder_sc(y, src_p, dst_p, pe)
    return out.reshape(T, D)


# speedup vs baseline: 1.0012x; 1.0012x over previous
import functools
import math

import jax
import jax.numpy as jnp
from jax import lax
from jax.experimental import pallas as pl
from jax.experimental.pallas import tpu as pltpu
from jax.experimental.pallas import tpu_sc as plsc

T = 16
N = 10000
E = 5000
D = 128

NC = 2
NS = 16
L = 16

EDGES_PER_TILE = 320
E_PAD = NS * EDGES_PER_TILE
CHUNK = 80
NCHUNK = EDGES_PER_TILE // CHUNK
ROWS_PER_TILE = 640
N_Y = NS * ROWS_PER_TILE
N_PAD = N_Y + L
SLICES_PER_SC = T // NC
RCHUNK = 80
NR = ROWS_PER_TILE // RCHUNK
NEG = -3.0e38


def _pe_table(t, d_model):
    position = jnp.arange(t, dtype=jnp.float32)[:, None]
    div_term = jnp.exp(
        jnp.arange(0, d_model, 2, dtype=jnp.float32) * (-math.log(10000.0) / d_model))
    pe = jnp.zeros((t, d_model), dtype=jnp.float32)
    pe = pe.at[:, 0::2].set(jnp.sin(position * div_term))
    pe = pe.at[:, 1::2].set(jnp.cos(position * div_term))
    return pe


def _matmul_body(q_ref, w_ref, y_ref):
    y_ref[...] = jnp.dot(q_ref[...], w_ref[...],
                         preferred_element_type=jnp.float32)


def _compute_y(q, w):
    blk = 1280
    return pl.pallas_call(
        _matmul_body,
        grid=(N_Y // blk,),
        in_specs=[pl.BlockSpec((blk, D), lambda i: (i, 0)),
                  pl.BlockSpec((D, D), lambda i: (0, 0))],
        out_specs=pl.BlockSpec((blk, D), lambda i: (i, 0)),
        out_shape=jax.ShapeDtypeStruct((N_Y, D), jnp.float32),
    )(q, w)


_mesh = plsc.VectorSubcoreMesh(core_axis_name="c", subcore_axis_name="s")


@functools.partial(
    pl.kernel,
    mesh=_mesh,
    out_type=jax.ShapeDtypeStruct((T * D,), jnp.float32),
    scratch_types=[
        pltpu.VMEM_SHARED((N_PAD, D), jnp.float32),
        pltpu.VMEM_SHARED((NS, D), jnp.float32),
        pltpu.VMEM((NCHUNK, CHUNK), jnp.int32),
        pltpu.VMEM((NCHUNK, CHUNK), jnp.int32),
        pltpu.VMEM((2, CHUNK, D), jnp.float32),
        pltpu.VMEM((2, RCHUNK, D), jnp.float32),
        pltpu.VMEM((D,), jnp.float32),
        pltpu.VMEM((NS, D), jnp.float32),
        pltpu.VMEM((D,), jnp.float32),
        pltpu.VMEM((D,), jnp.float32),
        pltpu.SemaphoreType.DMA,
        pltpu.SemaphoreType.DMA,
        pltpu.SemaphoreType.DMA,
        pltpu.SemaphoreType.DMA,
    ],
)
def _encoder_sc(y_hbm, src_hbm, dst_hbm, pe_hbm, out_hbm,
                agg_sh, partials_sh, src_idx_v, dst_idx_v, rows_v,
                stripe_v, vec_v, partials_v, pe_v, out_v, gsem, ssem, msem, isem):
    c = lax.axis_index("c")
    s = lax.axis_index("s")
    rbase = s * ROWS_PER_TILE

    t0 = c * SLICES_PER_SC
    pltpu.sync_copy(src_hbm.at[t0, s], src_idx_v)
    pltpu.sync_copy(dst_hbm.at[t0, s], dst_idx_v)
    pltpu.async_copy(y_hbm.at[src_idx_v.at[0]], rows_v.at[0], gsem)
    pltpu.sync_copy(y_hbm.at[pl.ds(rbase, ROWS_PER_TILE)],
                    agg_sh.at[pl.ds(rbase, ROWS_PER_TILE)])
    plsc.subcore_barrier()

    def slice_body(k, carry):
        t = c * SLICES_PER_SC + k
        for j in range(NCHUNK):
            pltpu.make_async_copy(y_hbm.at[src_idx_v.at[j]],
                                  rows_v.at[j % 2], gsem).wait()
            pltpu.async_copy(rows_v.at[j % 2],
                             agg_sh.at[dst_idx_v.at[j]], ssem, add=True)
            if j + 1 < NCHUNK:
                if j >= 1:
                    pltpu.make_async_copy(rows_v.at[(j - 1) % 2],
                                          agg_sh.at[dst_idx_v.at[j - 1]],
                                          ssem).wait()
                pltpu.async_copy(y_hbm.at[src_idx_v.at[j + 1]],
                                 rows_v.at[(j + 1) % 2], gsem)
        for j in (NCHUNK - 2, NCHUNK - 1):
            pltpu.make_async_copy(rows_v.at[j % 2],
                                  agg_sh.at[dst_idx_v.at[j]], ssem).wait()
        plsc.subcore_barrier()
        pltpu.async_copy(agg_sh.at[pl.ds(rbase, RCHUNK)], stripe_v.at[0], msem)

        @pl.when(k + 1 < SLICES_PER_SC)
        def _():
            tn = t + 1
            pltpu.sync_copy(src_hbm.at[tn, s], src_idx_v)
            pltpu.sync_copy(dst_hbm.at[tn, s], dst_idx_v)
            pltpu.async_copy(y_hbm.at[src_idx_v.at[0]], rows_v.at[0], gsem)

        acc = tuple(jnp.full((L,), NEG, jnp.float32) for _ in range(D // L))
        for cb in range(NR):
            pltpu.make_async_copy(
                agg_sh.at[pl.ds(rbase + cb * RCHUNK, RCHUNK)],
                stripe_v.at[cb % 2], msem).wait()
            if cb + 1 < NR:
                pltpu.async_copy(
                    agg_sh.at[pl.ds(rbase + (cb + 1) * RCHUNK, RCHUNK)],
                    stripe_v.at[(cb + 1) % 2], msem)

            @pl.when(k + 1 < SLICES_PER_SC)
            def _():
                pltpu.async_copy(
                    y_hbm.at[pl.ds(rbase + cb * RCHUNK, RCHUNK)],
                    agg_sh.at[pl.ds(rbase + cb * RCHUNK, RCHUNK)], isem)

            buf = cb % 2

            def row_body(r, a):
                return tuple(
                    jnp.maximum(a[j], stripe_v[buf, r, pl.ds(j * L, L)])
                    for j in range(D // L))

            acc = lax.fori_loop(0, RCHUNK, row_body, acc)

        @pl.when(k + 1 < SLICES_PER_SC)
        def _():
            for cb in range(NR):
                pltpu.make_async_copy(
                    y_hbm.at[pl.ds(rbase + cb * RCHUNK, RCHUNK)],
                    agg_sh.at[pl.ds(rbase + cb * RCHUNK, RCHUNK)],
                    isem).wait()

        for j in range(D // L):
            vec_v[pl.ds(j * L, L)] = jnp.maximum(acc[j], 0.0)
        pltpu.sync_copy(vec_v, partials_sh.at[s])
        plsc.subcore_barrier()

        @pl.when(s == 0)
        def _():
            pltpu.sync_copy(partials_sh, partials_v)
            pltpu.sync_copy(pe_hbm.at[pl.ds(t * D, D)], pe_v)
            for j in range(D // L):
                m = partials_v[0, pl.ds(j * L, L)]
                for i in range(1, NS):
                    m = jnp.maximum(m, partials_v[i, pl.ds(j * L, L)])
                out_v[pl.ds(j * L, L)] = m + pe_v[pl.ds(j * L, L)]
            pltpu.sync_copy(out_v, out_hbm.at[pl.ds(t * D, D)])

        plsc.subcore_barrier()
        return carry

    lax.fori_loop(0, SLICES_PER_SC, slice_body, 0)


def kernel(q_embeddings, W, edge_index):
    q_pad = jnp.concatenate(
        [q_embeddings, jnp.zeros((N_Y - N, D), jnp.float32)], axis=0)
    y = _compute_y(q_pad, W)
    ei = edge_index.astype(jnp.int32)
    src = ei[:, 0, :]
    dst = ei[:, 1, :]
    pad_src = jnp.zeros((T, E_PAD - E), jnp.int32)
    pad_dst = jnp.full((T, E_PAD - E), N_Y, jnp.int32)
    src_p = jnp.concatenate([src, pad_src], axis=1).reshape(T, NS, NCHUNK, CHUNK)
    dst_p = jnp.concatenate([dst, pad_dst], axis=1).reshape(T, NS, NCHUNK, CHUNK)
    pe = _pe_table(T, D).reshape(T * D)
    out = _encoder_sc(y, src_p, dst_p, pe)
    return out.reshape(T, D)


# speedup vs baseline: 1.0264x; 1.0252x over previous
import functools
import math

import jax
import jax.numpy as jnp
from jax import lax
from jax.experimental import pallas as pl
from jax.experimental.pallas import tpu as pltpu
from jax.experimental.pallas import tpu_sc as plsc

T = 16
N = 10000
E = 5000
D = 128

NC = 2
NS = 16
L = 16

EDGES_PER_TILE = 320
E_PAD = NS * EDGES_PER_TILE
CHUNK = 80
NCHUNK = EDGES_PER_TILE // CHUNK
ROWS_PER_TILE = 640
N_Y = NS * ROWS_PER_TILE
N_PAD = N_Y + L
SLICES_PER_SC = T // NC
RCHUNK = 80
NR = ROWS_PER_TILE // RCHUNK
NEG = -3.0e38


def _pe_table(t, d_model):
    position = jnp.arange(t, dtype=jnp.float32)[:, None]
    div_term = jnp.exp(
        jnp.arange(0, d_model, 2, dtype=jnp.float32) * (-math.log(10000.0) / d_model))
    pe = jnp.zeros((t, d_model), dtype=jnp.float32)
    pe = pe.at[:, 0::2].set(jnp.sin(position * div_term))
    pe = pe.at[:, 1::2].set(jnp.cos(position * div_term))
    return pe


def _matmul_body(q_ref, w_ref, y_ref):
    y_ref[...] = jnp.dot(q_ref[...], w_ref[...],
                         preferred_element_type=jnp.float32)


def _compute_y(q, w):
    blk = 1280
    return pl.pallas_call(
        _matmul_body,
        grid=(N_Y // blk,),
        in_specs=[pl.BlockSpec((blk, D), lambda i: (i, 0)),
                  pl.BlockSpec((D, D), lambda i: (0, 0))],
        out_specs=pl.BlockSpec((blk, D), lambda i: (i, 0)),
        out_shape=jax.ShapeDtypeStruct((N_Y, D), jnp.float32),
    )(q, w)


_mesh = plsc.VectorSubcoreMesh(core_axis_name="c", subcore_axis_name="s")


@functools.partial(
    pl.kernel,
    mesh=_mesh,
    out_type=jax.ShapeDtypeStruct((T * D,), jnp.float32),
    scratch_types=[
        pltpu.VMEM_SHARED((N_PAD, D), jnp.float32),
        pltpu.VMEM_SHARED((NS, D), jnp.float32),
        pltpu.VMEM((NCHUNK, CHUNK), jnp.int32),
        pltpu.VMEM((NCHUNK, CHUNK), jnp.int32),
        pltpu.VMEM((2, CHUNK, D), jnp.float32),
        pltpu.VMEM((2, RCHUNK, D), jnp.float32),
        pltpu.VMEM((D,), jnp.float32),
        pltpu.VMEM((NS, D), jnp.float32),
        pltpu.VMEM((D,), jnp.float32),
        pltpu.VMEM((D,), jnp.float32),
        pltpu.SemaphoreType.DMA,
        pltpu.SemaphoreType.DMA,
        pltpu.SemaphoreType.DMA,
        pltpu.SemaphoreType.DMA,
    ],
)
def _encoder_sc(y_hbm, src_hbm, dst_hbm, pe_hbm, out_hbm,
                agg_sh, partials_sh, src_idx_v, dst_idx_v, rows_v,
                stripe_v, vec_v, partials_v, pe_v, out_v, gsem, ssem, msem, isem):
    c = lax.axis_index("c")
    s = lax.axis_index("s")
    rbase = s * ROWS_PER_TILE

    t0 = c * SLICES_PER_SC
    pltpu.sync_copy(src_hbm.at[t0, s], src_idx_v)
    pltpu.sync_copy(dst_hbm.at[t0, s], dst_idx_v)
    pltpu.async_copy(y_hbm.at[src_idx_v.at[0]], rows_v.at[0], gsem)
    pltpu.sync_copy(y_hbm.at[pl.ds(rbase, ROWS_PER_TILE)],
                    agg_sh.at[pl.ds(rbase, ROWS_PER_TILE)])
    plsc.subcore_barrier()

    def slice_body(k, carry):
        t = c * SLICES_PER_SC + k
        for j in range(NCHUNK):
            pltpu.make_async_copy(y_hbm.at[src_idx_v.at[j]],
                                  rows_v.at[j % 2], gsem).wait()
            pltpu.async_copy(rows_v.at[j % 2],
                             agg_sh.at[dst_idx_v.at[j]], ssem, add=True)
            if j + 1 < NCHUNK:
                if j >= 1:
                    pltpu.make_async_copy(rows_v.at[(j - 1) % 2],
                                          agg_sh.at[dst_idx_v.at[j - 1]],
                                          ssem).wait()
                pltpu.async_copy(y_hbm.at[src_idx_v.at[j + 1]],
                                 rows_v.at[(j + 1) % 2], gsem)
        for j in (NCHUNK - 2, NCHUNK - 1):
            pltpu.make_async_copy(rows_v.at[j % 2],
                                  agg_sh.at[dst_idx_v.at[j]], ssem).wait()
        plsc.subcore_barrier()
        pltpu.async_copy(agg_sh.at[pl.ds(rbase, RCHUNK)], stripe_v.at[0], msem)

        @pl.when(k + 1 < SLICES_PER_SC)
        def _():
            tn = t + 1
            pltpu.sync_copy(src_hbm.at[tn, s], src_idx_v)
            pltpu.sync_copy(dst_hbm.at[tn, s], dst_idx_v)
            pltpu.async_copy(y_hbm.at[src_idx_v.at[0]], rows_v.at[0], gsem)

        acc = tuple(jnp.full((L,), NEG, jnp.float32) for _ in range(D // L))
        for cb in range(NR):
            pltpu.make_async_copy(
                agg_sh.at[pl.ds(rbase + cb * RCHUNK, RCHUNK)],
                stripe_v.at[cb % 2], msem).wait()
            if cb + 1 < NR:
                pltpu.async_copy(
                    agg_sh.at[pl.ds(rbase + (cb + 1) * RCHUNK, RCHUNK)],
                    stripe_v.at[(cb + 1) % 2], msem)

            @pl.when(k + 1 < SLICES_PER_SC)
            def _():
                pltpu.async_copy(
                    y_hbm.at[pl.ds(rbase + cb * RCHUNK, RCHUNK)],
                    agg_sh.at[pl.ds(rbase + cb * RCHUNK, RCHUNK)], isem)

            buf = cb % 2

            def row_body(r, a):
                return tuple(
                    jnp.maximum(a[j], stripe_v[buf, r, pl.ds(j * L, L)])
                    for j in range(D // L))

            acc = lax.fori_loop(0, RCHUNK, row_body, acc)

        @pl.when(k + 1 < SLICES_PER_SC)
        def _():
            for cb in range(NR):
                pltpu.make_async_copy(
                    y_hbm.at[pl.ds(rbase + cb * RCHUNK, RCHUNK)],
                    agg_sh.at[pl.ds(rbase + cb * RCHUNK, RCHUNK)],
                    isem).wait()

        for j in range(D // L):
            vec_v[pl.ds(j * L, L)] = jnp.maximum(acc[j], 0.0)
        pltpu.sync_copy(vec_v, partials_sh.at[s])
        plsc.subcore_barrier()

        @pl.when(s == 0)
        def _():
            pltpu.sync_copy(partials_sh, partials_v)
            pltpu.sync_copy(pe_hbm.at[pl.ds(t * D, D)], pe_v)
            for j in range(D // L):
                m = partials_v[0, pl.ds(j * L, L)]
                for i in range(1, NS):
                    m = jnp.maximum(m, partials_v[i, pl.ds(j * L, L)])
                out_v[pl.ds(j * L, L)] = m + pe_v[pl.ds(j * L, L)]
            pltpu.sync_copy(out_v, out_hbm.at[pl.ds(t * D, D)])

        return carry

    lax.fori_loop(0, SLICES_PER_SC, slice_body, 0)


def kernel(q_embeddings, W, edge_index):
    q_pad = jnp.concatenate(
        [q_embeddings, jnp.zeros((N_Y - N, D), jnp.float32)], axis=0)
    y = _compute_y(q_pad, W)
    ei = edge_index.astype(jnp.int32)
    src = ei[:, 0, :]
    dst = ei[:, 1, :]
    pad_src = jnp.zeros((T, E_PAD - E), jnp.int32)
    pad_dst = jnp.full((T, E_PAD - E), N_Y, jnp.int32)
    src_p = jnp.concatenate([src, pad_src], axis=1).reshape(T, NS, NCHUNK, CHUNK)
    dst_p = jnp.concatenate([dst, pad_dst], axis=1).reshape(T, NS, NCHUNK, CHUNK)
    pe = _pe_table(T, D).reshape(T * D)
    out = _encoder_sc(y, src_p, dst_p, pe)
    return out.reshape(T, D)


# speedup vs baseline: 1.0375x; 1.0108x over previous
import functools
import math

import jax
import jax.numpy as jnp
from jax import lax
from jax.experimental import pallas as pl
from jax.experimental.pallas import tpu as pltpu
from jax.experimental.pallas import tpu_sc as plsc

T = 16
N = 10000
E = 5000
D = 128

NC = 2
NS = 16
L = 16

EDGES_PER_TILE = 320
E_PAD = NS * EDGES_PER_TILE
CHUNK = 80
NCHUNK = EDGES_PER_TILE // CHUNK
ROWS_PER_TILE = 640
N_Y = NS * ROWS_PER_TILE
N_PAD = N_Y + L
SLICES_PER_SC = T // NC
RCHUNK = 80
NR = ROWS_PER_TILE // RCHUNK
NEG = -3.0e38


def _pe_table(t, d_model):
    position = jnp.arange(t, dtype=jnp.float32)[:, None]
    div_term = jnp.exp(
        jnp.arange(0, d_model, 2, dtype=jnp.float32) * (-math.log(10000.0) / d_model))
    pe = jnp.zeros((t, d_model), dtype=jnp.float32)
    pe = pe.at[:, 0::2].set(jnp.sin(position * div_term))
    pe = pe.at[:, 1::2].set(jnp.cos(position * div_term))
    return pe


def _matmul_body(q_ref, w_ref, y_ref):
    y_ref[...] = jnp.dot(q_ref[...], w_ref[...],
                         preferred_element_type=jnp.float32)


def _compute_y(q, w):
    blk = 1280
    return pl.pallas_call(
        _matmul_body,
        grid=(N_Y // blk,),
        in_specs=[pl.BlockSpec((blk, D), lambda i: (i, 0)),
                  pl.BlockSpec((D, D), lambda i: (0, 0))],
        out_specs=pl.BlockSpec((blk, D), lambda i: (i, 0)),
        out_shape=jax.ShapeDtypeStruct((N_Y, D), jnp.float32),
    )(q, w)


_mesh = plsc.VectorSubcoreMesh(core_axis_name="c", subcore_axis_name="s")


@functools.partial(
    pl.kernel,
    mesh=_mesh,
    out_type=jax.ShapeDtypeStruct((T * D,), jnp.float32),
    scratch_types=[
        pltpu.VMEM_SHARED((N_PAD, D), jnp.float32),
        pltpu.VMEM_SHARED((NS, D), jnp.float32),
        pltpu.VMEM((NCHUNK, CHUNK), jnp.int32),
        pltpu.VMEM((NCHUNK, CHUNK), jnp.int32),
        pltpu.VMEM((2, CHUNK, D), jnp.float32),
        pltpu.VMEM((2, RCHUNK, D), jnp.float32),
        pltpu.VMEM((D,), jnp.float32),
        pltpu.VMEM((NS, D), jnp.float32),
        pltpu.VMEM((D,), jnp.float32),
        pltpu.VMEM((D,), jnp.float32),
        pltpu.SemaphoreType.DMA,
        pltpu.SemaphoreType.DMA,
        pltpu.SemaphoreType.DMA,
        pltpu.SemaphoreType.DMA,
    ],
)
def _encoder_sc(y_hbm, src_hbm, dst_hbm, pe_hbm, out_hbm,
                agg_sh, partials_sh, src_idx_v, dst_idx_v, rows_v,
                stripe_v, vec_v, partials_v, pe_v, out_v, gsem, ssem, msem, isem):
    c = lax.axis_index("c")
    s = lax.axis_index("s")
    rbase = s * ROWS_PER_TILE

    t0 = c * SLICES_PER_SC
    pltpu.sync_copy(src_hbm.at[t0, s], src_idx_v)
    pltpu.sync_copy(dst_hbm.at[t0, s], dst_idx_v)
    pltpu.async_copy(y_hbm.at[src_idx_v.at[0]], rows_v.at[0], gsem)
    pltpu.sync_copy(y_hbm.at[pl.ds(rbase, ROWS_PER_TILE)],
                    agg_sh.at[pl.ds(rbase, ROWS_PER_TILE)])
    plsc.subcore_barrier()

    def slice_body(k, carry):
        t = c * SLICES_PER_SC + k
        for j in range(NCHUNK):
            pltpu.make_async_copy(y_hbm.at[src_idx_v.at[j]],
                                  rows_v.at[j % 2], gsem).wait()
            pltpu.async_copy(rows_v.at[j % 2],
                             agg_sh.at[dst_idx_v.at[j]], ssem, add=True)
            if j + 1 < NCHUNK:
                if j >= 1:
                    pltpu.make_async_copy(rows_v.at[(j - 1) % 2],
                                          agg_sh.at[dst_idx_v.at[j - 1]],
                                          ssem).wait()
                pltpu.async_copy(y_hbm.at[src_idx_v.at[j + 1]],
                                 rows_v.at[(j + 1) % 2], gsem)
        for j in (NCHUNK - 2, NCHUNK - 1):
            pltpu.make_async_copy(rows_v.at[j % 2],
                                  agg_sh.at[dst_idx_v.at[j]], ssem).wait()
        plsc.subcore_barrier()
        pltpu.async_copy(agg_sh.at[pl.ds(rbase, RCHUNK)], stripe_v.at[0], msem)

        @pl.when(k + 1 < SLICES_PER_SC)
        def _():
            tn = t + 1
            pltpu.sync_copy(src_hbm.at[tn, s], src_idx_v)
            pltpu.sync_copy(dst_hbm.at[tn, s], dst_idx_v)
            pltpu.async_copy(y_hbm.at[src_idx_v.at[0]], rows_v.at[0], gsem)

        acc = tuple(jnp.full((L,), NEG, jnp.float32) for _ in range(D // L))
        for cb in range(NR):
            pltpu.make_async_copy(
                agg_sh.at[pl.ds(rbase + cb * RCHUNK, RCHUNK)],
                stripe_v.at[cb % 2], msem).wait()
            if cb + 1 < NR:
                pltpu.async_copy(
                    agg_sh.at[pl.ds(rbase + (cb + 1) * RCHUNK, RCHUNK)],
                    stripe_v.at[(cb + 1) % 2], msem)

            @pl.when(k + 1 < SLICES_PER_SC)
            def _():
                pltpu.async_copy(
                    y_hbm.at[pl.ds(rbase + cb * RCHUNK, RCHUNK)],
                    agg_sh.at[pl.ds(rbase + cb * RCHUNK, RCHUNK)], isem)

            buf = cb % 2

            def row_body(r4, a):
                r = r4 * 4
                for u in range(4):
                    a = tuple(
                        jnp.maximum(a[j], stripe_v[buf, r + u, pl.ds(j * L, L)])
                        for j in range(D // L))
                return a

            acc = lax.fori_loop(0, RCHUNK // 4, row_body, acc)

        @pl.when(k + 1 < SLICES_PER_SC)
        def _():
            for cb in range(NR):
                pltpu.make_async_copy(
                    y_hbm.at[pl.ds(rbase + cb * RCHUNK, RCHUNK)],
                    agg_sh.at[pl.ds(rbase + cb * RCHUNK, RCHUNK)],
                    isem).wait()

        for j in range(D // L):
            vec_v[pl.ds(j * L, L)] = jnp.maximum(acc[j], 0.0)
        pltpu.sync_copy(vec_v, partials_sh.at[s])
        plsc.subcore_barrier()

        @pl.when(s == 0)
        def _():
            pltpu.sync_copy(partials_sh, partials_v)
            pltpu.sync_copy(pe_hbm.at[pl.ds(t * D, D)], pe_v)
            for j in range(D // L):
                m = partials_v[0, pl.ds(j * L, L)]
                for i in range(1, NS):
                    m = jnp.maximum(m, partials_v[i, pl.ds(j * L, L)])
                out_v[pl.ds(j * L, L)] = m + pe_v[pl.ds(j * L, L)]
            pltpu.sync_copy(out_v, out_hbm.at[pl.ds(t * D, D)])

        return carry

    lax.fori_loop(0, SLICES_PER_SC, slice_body, 0)


def kernel(q_embeddings, W, edge_index):
    q_pad = jnp.concatenate(
        [q_embeddings, jnp.zeros((N_Y - N, D), jnp.float32)], axis=0)
    y = _compute_y(q_pad, W)
    ei = edge_index.astype(jnp.int32)
    src = ei[:, 0, :]
    dst = ei[:, 1, :]
    pad_src = jnp.zeros((T, E_PAD - E), jnp.int32)
    pad_dst = jnp.full((T, E_PAD - E), N_Y, jnp.int32)
    src_p = jnp.concatenate([src, pad_src], axis=1).reshape(T, NS, NCHUNK, CHUNK)
    dst_p = jnp.concatenate([dst, pad_dst], axis=1).reshape(T, NS, NCHUNK, CHUNK)
    pe = _pe_table(T, D).reshape(T * D)
    out = _encoder_sc(y, src_p, dst_p, pe)
    return out.reshape(T, D)
